# initial kernel scaffold (unmeasured)
import os

import jax
import jax.numpy as jnp
from jax import lax
from jax.experimental import pallas as pl
from jax.experimental.pallas import tpu as pltpu

N_DEV = 8
M_BLK = 512
N_COLS = 8192
N_HALF = 4096
K_SHARD = 512

_INTERPRET = os.environ.get("KERNEL_INTERPRET") == "1"


def kernel(x, w_mat, scale_x, scale_w):
    def body(x_ref, w_ref, sx_ref, sw_ref, out_ref,
             acc_cw, acc_ccw, rbuf_cw, rbuf_ccw,
             send_cw_sem, send_ccw_sem, recv_cw_sem, recv_ccw_sem,
             credit_cw, credit_ccw):
        me = lax.axis_index("i")
        right = lax.rem(me + 1, N_DEV)
        left = lax.rem(me + N_DEV - 1, N_DEV)

        bar = pltpu.get_barrier_semaphore()
        for nbr in (left, right):
            pl.semaphore_signal(bar, inc=1, device_id=(nbr,),
                                device_id_type=pl.DeviceIdType.MESH)
        pl.semaphore_wait(bar, 2)

        scale = sx_ref[0] * sw_ref[0]

        def local_block(c, lo):
            xs = x_ref[pl.ds(c * M_BLK, M_BLK), :]
            ws = w_ref[:, lo:lo + N_HALF]
            return lax.dot_general(xs, ws, (((1,), (0,)), ((), ())),
                                   preferred_element_type=jnp.int32)

        def recv_desc(rbuf, rsem, slot, src_buf, ssem, nbr):
            return pltpu.make_async_remote_copy(
                src_ref=src_buf.at[slot], dst_ref=rbuf.at[slot],
                send_sem=ssem.at[slot], recv_sem=rsem.at[slot],
                device_id=(nbr,), device_id_type=pl.DeviceIdType.MESH)

        for s in range(N_DEV):
            slot = s % 2
            pslot = (s - 1) % 2
            c_cw = lax.rem(me + (2 * N_DEV - 1 - s), N_DEV)
            c_ccw = lax.rem(me + (1 + s), N_DEV)

            a_cw = local_block(c_cw, 0)
            a_ccw = local_block(c_ccw, N_HALF)

            if s > 0:
                rcv_cw = recv_desc(rbuf_cw, recv_cw_sem, pslot,
                                   acc_cw, send_cw_sem, left)
                rcv_cw.wait_recv()
                a_cw = a_cw + rbuf_cw[pslot]
                rcv_ccw = recv_desc(rbuf_ccw, recv_ccw_sem, pslot,
                                    acc_ccw, send_ccw_sem, right)
                rcv_ccw.wait_recv()
                a_ccw = a_ccw + rbuf_ccw[pslot]

            if s < N_DEV - 1:
                acc_cw[slot] = a_cw
                acc_ccw[slot] = a_ccw
                if 1 <= s <= N_DEV - 3:
                    pl.semaphore_signal(credit_cw, inc=1, device_id=(left,),
                                        device_id_type=pl.DeviceIdType.MESH)
                    pl.semaphore_signal(credit_ccw, inc=1, device_id=(right,),
                                        device_id_type=pl.DeviceIdType.MESH)
                if s >= 2:
                    pl.semaphore_wait(credit_cw, 1)
                    pl.semaphore_wait(credit_ccw, 1)
                snd_cw = pltpu.make_async_remote_copy(
                    src_ref=acc_cw.at[slot], dst_ref=rbuf_cw.at[slot],
                    send_sem=send_cw_sem.at[slot], recv_sem=recv_cw_sem.at[slot],
                    device_id=(right,), device_id_type=pl.DeviceIdType.MESH)
                snd_ccw = pltpu.make_async_remote_copy(
                    src_ref=acc_ccw.at[slot], dst_ref=rbuf_ccw.at[slot],
                    send_sem=send_ccw_sem.at[slot], recv_sem=recv_ccw_sem.at[slot],
                    device_id=(left,), device_id_type=pl.DeviceIdType.MESH)
                snd_cw.start()
                snd_ccw.start()
                snd_cw.wait_send()
                snd_ccw.wait_send()
            else:
                out_ref[:, :N_HALF] = jnp.maximum(
                    a_cw.astype(jnp.float32) * scale, 0.0)
                out_ref[:, N_HALF:] = jnp.maximum(
                    a_ccw.astype(jnp.float32) * scale, 0.0)

    return pl.pallas_call(
        body,
        out_shape=jax.ShapeDtypeStruct((M_BLK, N_COLS), jnp.float32),
        in_specs=[
            pl.BlockSpec(memory_space=pltpu.VMEM),
            pl.BlockSpec(memory_space=pltpu.VMEM),
            pl.BlockSpec(memory_space=pltpu.SMEM),
            pl.BlockSpec(memory_space=pltpu.SMEM),
        ],
        out_specs=pl.BlockSpec(memory_space=pltpu.VMEM),
        scratch_shapes=[
            pltpu.VMEM((2, M_BLK, N_HALF), jnp.int32),
            pltpu.VMEM((2, M_BLK, N_HALF), jnp.int32),
            pltpu.VMEM((2, M_BLK, N_HALF), jnp.int32),
            pltpu.VMEM((2, M_BLK, N_HALF), jnp.int32),
            pltpu.SemaphoreType.DMA((2,)),
            pltpu.SemaphoreType.DMA((2,)),
            pltpu.SemaphoreType.DMA((2,)),
            pltpu.SemaphoreType.DMA((2,)),
            pltpu.SemaphoreType.REGULAR,
            pltpu.SemaphoreType.REGULAR,
        ],
        compiler_params=pltpu.CompilerParams(collective_id=0),
        interpret=_INTERPRET,
    )(x, w_mat, scale_x, scale_w)


# baseline (device time: 706995 ns/iter reference)
import os

import jax
import jax.numpy as jnp
from jax import lax
from jax.experimental import pallas as pl
from jax.experimental.pallas import tpu as pltpu

N_DEV = 8
M_BLK = 512
N_COLS = 8192
N_HALF = 4096
N_SUB = 2
C_W = N_HALF // N_SUB

_INTERPRET = os.environ.get("KERNEL_INTERPRET") == "1"


def kernel(x, w_mat, scale_x, scale_w):
    def body(x_ref, w_ref, sx_ref, sw_ref, out_ref,
             acc_cw, acc_ccw, rbuf_cw, rbuf_ccw,
             send_cw_sem, send_ccw_sem, recv_cw_sem, recv_ccw_sem,
             credit_cw, credit_ccw):
        me = lax.axis_index("i")
        right = lax.rem(me + 1, N_DEV)
        left = lax.rem(me + N_DEV - 1, N_DEV)

        bar = pltpu.get_barrier_semaphore()
        for nbr in (left, right):
            pl.semaphore_signal(bar, inc=1, device_id=nbr,
                                device_id_type=pl.DeviceIdType.LOGICAL)
        pl.semaphore_wait(bar, 2)

        scale = sx_ref[0] * sw_ref[0]

        def desc(acc, rbuf, ssem, rsem, q, nbr):
            return pltpu.make_async_remote_copy(
                src_ref=acc.at[q], dst_ref=rbuf.at[q],
                send_sem=ssem.at[q], recv_sem=rsem.at[q],
                device_id=nbr, device_id_type=pl.DeviceIdType.LOGICAL)

        channels = [
            (acc_cw, rbuf_cw, send_cw_sem, recv_cw_sem, credit_cw,
             right, left, 0),
            (acc_ccw, rbuf_ccw, send_ccw_sem, recv_ccw_sem, credit_ccw,
             left, right, N_HALF),
        ]

        for s in range(N_DEV):
            c_cw = lax.rem(me + (2 * N_DEV - 1 - s), N_DEV)
            c_ccw = lax.rem(me + (1 + s), N_DEV)
            blocks = (c_cw, c_ccw)

            for d, (acc, rbuf, ssem, rsem, credit, dst, ups, col0) in \
                    enumerate(channels):
                xs = x_ref[pl.ds(blocks[d] * M_BLK, M_BLK), :]
                for q in range(N_SUB):
                    lo = col0 + q * C_W
                    a = lax.dot_general(
                        xs, w_ref[:, lo:lo + C_W], (((1,), (0,)), ((), ())),
                        preferred_element_type=jnp.int32)
                    if s > 0:
                        desc(acc, rbuf, ssem, rsem, q, ups).wait_recv()
                        a = a + rbuf[q]
                    if s < N_DEV - 1:
                        acc[q] = a
                        if 1 <= s <= N_DEV - 2:
                            pl.semaphore_signal(
                                credit.at[q], inc=1, device_id=ups,
                                device_id_type=pl.DeviceIdType.LOGICAL)
                    else:
                        out_ref[:, lo:lo + C_W] = jnp.maximum(
                            a.astype(jnp.float32) * scale, 0.0)

            if s < N_DEV - 1:
                sends = []
                for acc, rbuf, ssem, rsem, credit, dst, ups, col0 in channels:
                    for q in range(N_SUB):
                        if s >= 1:
                            pl.semaphore_wait(credit.at[q], 1)
                        snd = desc(acc, rbuf, ssem, rsem, q, dst)
                        snd.start()
                        sends.append(snd)
                if not _INTERPRET:
                    for snd in sends:
                        snd.wait_send()

    return pl.pallas_call(
        body,
        out_shape=jax.ShapeDtypeStruct((M_BLK, N_COLS), jnp.float32),
        in_specs=[
            pl.BlockSpec(memory_space=pltpu.VMEM),
            pl.BlockSpec(memory_space=pltpu.VMEM),
            pl.BlockSpec(memory_space=pltpu.SMEM),
            pl.BlockSpec(memory_space=pltpu.SMEM),
        ],
        out_specs=pl.BlockSpec(memory_space=pltpu.VMEM),
        scratch_shapes=[
            pltpu.VMEM((N_SUB, M_BLK, C_W), jnp.int32),
            pltpu.VMEM((N_SUB, M_BLK, C_W), jnp.int32),
            pltpu.VMEM((N_SUB, M_BLK, C_W), jnp.int32),
            pltpu.VMEM((N_SUB, M_BLK, C_W), jnp.int32),
            pltpu.SemaphoreType.DMA((N_SUB,)),
            pltpu.SemaphoreType.DMA((N_SUB,)),
            pltpu.SemaphoreType.DMA((N_SUB,)),
            pltpu.SemaphoreType.DMA((N_SUB,)),
            pltpu.SemaphoreType.REGULAR((N_SUB,)),
            pltpu.SemaphoreType.REGULAR((N_SUB,)),
        ],
        compiler_params=pltpu.CompilerParams(
            collective_id=0, vmem_limit_bytes=62 * 1024 * 1024),
        interpret=_INTERPRET,
    )(x, w_mat, scale_x, scale_w)


# device time: 663230 ns/iter; 1.0660x vs baseline; 1.0660x over previous
import os

import jax
import jax.numpy as jnp
from jax import lax
from jax.experimental import pallas as pl
from jax.experimental.pallas import tpu as pltpu

N_DEV = 8
M_BLK = 512
N_COLS = 8192
N_HALF = 4096
N_SUB = 2
C_W = N_HALF // N_SUB

_INTERPRET = os.environ.get("KERNEL_INTERPRET") == "1"


def kernel(x, w_mat, scale_x, scale_w):
    def body(x_ref, w_ref, sx_ref, sw_ref, out_ref,
             acc_cw, acc_ccw, rbuf_cw, rbuf_ccw,
             send_cw_sem, send_ccw_sem, recv_cw_sem, recv_ccw_sem,
             credit_cw, credit_ccw):
        me = lax.axis_index("i")
        right = lax.rem(me + 1, N_DEV)
        left = lax.rem(me + N_DEV - 1, N_DEV)

        bar = pltpu.get_barrier_semaphore()
        for nbr in (left, right):
            pl.semaphore_signal(bar, inc=1, device_id=nbr,
                                device_id_type=pl.DeviceIdType.LOGICAL)
        pl.semaphore_wait(bar, 2)

        scale = sx_ref[0] * sw_ref[0]

        def desc(acc, rbuf, ssem, rsem, q, nbr):
            return pltpu.make_async_remote_copy(
                src_ref=acc.at[q], dst_ref=rbuf.at[q],
                send_sem=ssem.at[q], recv_sem=rsem.at[q],
                device_id=nbr, device_id_type=pl.DeviceIdType.LOGICAL)

        groups = []
        for q in range(N_SUB):
            groups.append([
                (acc_cw, rbuf_cw, send_cw_sem, recv_cw_sem,
                 credit_cw, right, left, q * C_W, q, 0),
                (acc_ccw, rbuf_ccw, send_ccw_sem, recv_ccw_sem,
                 credit_ccw, left, right, N_HALF + q * C_W, q, 1),
            ])

        for s in range(N_DEV):
            c_cw = lax.rem(me + (2 * N_DEV - 1 - s), N_DEV)
            c_ccw = lax.rem(me + (1 + s), N_DEV)
            blocks = (c_cw, c_ccw)

            for group in groups:
                for acc, rbuf, ssem, rsem, credit, dst, ups, lo, q, d in group:
                    if s > 0 and not _INTERPRET:
                        desc(acc, rbuf, ssem, rsem, q, dst).wait_send()
                    xs = x_ref[pl.ds(blocks[d] * M_BLK, M_BLK), :]
                    a = lax.dot_general(
                        xs, w_ref[:, lo:lo + C_W], (((1,), (0,)), ((), ())),
                        preferred_element_type=jnp.int32)
                    if s > 0:
                        desc(acc, rbuf, ssem, rsem, q, ups).wait_recv()
                        a = a + rbuf[q]
                    if s < N_DEV - 1:
                        acc[q] = a
                        if s >= 1:
                            pl.semaphore_signal(
                                credit.at[q], inc=1, device_id=ups,
                                device_id_type=pl.DeviceIdType.LOGICAL)
                    else:
                        out_ref[:, lo:lo + C_W] = jnp.maximum(
                            a.astype(jnp.float32) * scale, 0.0)
                if s < N_DEV - 1:
                    for acc, rbuf, ssem, rsem, credit, dst, ups, lo, q, d \
                            in group:
                        if s >= 1:
                            pl.semaphore_wait(credit.at[q], 1)
                        desc(acc, rbuf, ssem, rsem, q, dst).start()

    return pl.pallas_call(
        body,
        out_shape=jax.ShapeDtypeStruct((M_BLK, N_COLS), jnp.float32),
        in_specs=[
            pl.BlockSpec(memory_space=pltpu.VMEM),
            pl.BlockSpec(memory_space=pltpu.VMEM),
            pl.BlockSpec(memory_space=pltpu.SMEM),
            pl.BlockSpec(memory_space=pltpu.SMEM),
        ],
        out_specs=pl.BlockSpec(memory_space=pltpu.VMEM),
        scratch_shapes=[
            pltpu.VMEM((N_SUB, M_BLK, C_W), jnp.int32),
            pltpu.VMEM((N_SUB, M_BLK, C_W), jnp.int32),
            pltpu.VMEM((N_SUB, M_BLK, C_W), jnp.int32),
            pltpu.VMEM((N_SUB, M_BLK, C_W), jnp.int32),
            pltpu.SemaphoreType.DMA((N_SUB,)),
            pltpu.SemaphoreType.DMA((N_SUB,)),
            pltpu.SemaphoreType.DMA((N_SUB,)),
            pltpu.SemaphoreType.DMA((N_SUB,)),
            pltpu.SemaphoreType.REGULAR((N_SUB,)),
            pltpu.SemaphoreType.REGULAR((N_SUB,)),
        ],
        compiler_params=pltpu.CompilerParams(
            collective_id=0, vmem_limit_bytes=62 * 1024 * 1024),
        interpret=_INTERPRET,
    )(x, w_mat, scale_x, scale_w)


# device time: 660577 ns/iter; 1.0703x vs baseline; 1.0040x over previous
import os

import jax
import jax.numpy as jnp
from jax import lax
from jax.experimental import pallas as pl
from jax.experimental.pallas import tpu as pltpu

N_DEV = 8
M_BLK = 512
N_COLS = 8192
N_HALF = 4096
N_SUB = 4
C_W = N_HALF // N_SUB

_INTERPRET = os.environ.get("KERNEL_INTERPRET") == "1"


def kernel(x, w_mat, scale_x, scale_w):
    def body(x_ref, w_ref, sx_ref, sw_ref, out_ref,
             acc_cw, acc_ccw, rbuf_cw, rbuf_ccw,
             send_cw_sem, send_ccw_sem, recv_cw_sem, recv_ccw_sem,
             credit_cw, credit_ccw):
        me = lax.axis_index("i")
        right = lax.rem(me + 1, N_DEV)
        left = lax.rem(me + N_DEV - 1, N_DEV)

        bar = pltpu.get_barrier_semaphore()
        for nbr in (left, right):
            pl.semaphore_signal(bar, inc=1, device_id=nbr,
                                device_id_type=pl.DeviceIdType.LOGICAL)
        pl.semaphore_wait(bar, 2)

        scale = sx_ref[0] * sw_ref[0]

        def desc(acc, rbuf, ssem, rsem, q, nbr):
            return pltpu.make_async_remote_copy(
                src_ref=acc.at[q], dst_ref=rbuf.at[q],
                send_sem=ssem.at[q], recv_sem=rsem.at[q],
                device_id=nbr, device_id_type=pl.DeviceIdType.LOGICAL)

        groups = []
        for q in range(N_SUB):
            groups.append([
                (acc_cw, rbuf_cw, send_cw_sem, recv_cw_sem,
                 credit_cw, right, left, q * C_W, q, 0),
                (acc_ccw, rbuf_ccw, send_ccw_sem, recv_ccw_sem,
                 credit_ccw, left, right, N_HALF + q * C_W, q, 1),
            ])

        for s in range(N_DEV):
            c_cw = lax.rem(me + (2 * N_DEV - 1 - s), N_DEV)
            c_ccw = lax.rem(me + (1 + s), N_DEV)
            blocks = (c_cw, c_ccw)

            for group in groups:
                for acc, rbuf, ssem, rsem, credit, dst, ups, lo, q, d in group:
                    if s > 0 and not _INTERPRET:
                        desc(acc, rbuf, ssem, rsem, q, dst).wait_send()
                    xs = x_ref[pl.ds(blocks[d] * M_BLK, M_BLK), :]
                    a = lax.dot_general(
                        xs, w_ref[:, lo:lo + C_W], (((1,), (0,)), ((), ())),
                        preferred_element_type=jnp.int32)
                    if s > 0:
                        desc(acc, rbuf, ssem, rsem, q, ups).wait_recv()
                        a = a + rbuf[q]
                    if s < N_DEV - 1:
                        acc[q] = a
                        if s >= 1:
                            pl.semaphore_signal(
                                credit.at[q], inc=1, device_id=ups,
                                device_id_type=pl.DeviceIdType.LOGICAL)
                    else:
                        out_ref[:, lo:lo + C_W] = jnp.maximum(
                            a.astype(jnp.float32) * scale, 0.0)
                if s < N_DEV - 1:
                    for acc, rbuf, ssem, rsem, credit, dst, ups, lo, q, d \
                            in group:
                        if s >= 1:
                            pl.semaphore_wait(credit.at[q], 1)
                        desc(acc, rbuf, ssem, rsem, q, dst).start()

    return pl.pallas_call(
        body,
        out_shape=jax.ShapeDtypeStruct((M_BLK, N_COLS), jnp.float32),
        in_specs=[
            pl.BlockSpec(memory_space=pltpu.VMEM),
            pl.BlockSpec(memory_space=pltpu.VMEM),
            pl.BlockSpec(memory_space=pltpu.SMEM),
            pl.BlockSpec(memory_space=pltpu.SMEM),
        ],
        out_specs=pl.BlockSpec(memory_space=pltpu.VMEM),
        scratch_shapes=[
            pltpu.VMEM((N_SUB, M_BLK, C_W), jnp.int32),
            pltpu.VMEM((N_SUB, M_BLK, C_W), jnp.int32),
            pltpu.VMEM((N_SUB, M_BLK, C_W), jnp.int32),
            pltpu.VMEM((N_SUB, M_BLK, C_W), jnp.int32),
            pltpu.SemaphoreType.DMA((N_SUB,)),
            pltpu.SemaphoreType.DMA((N_SUB,)),
            pltpu.SemaphoreType.DMA((N_SUB,)),
            pltpu.SemaphoreType.DMA((N_SUB,)),
            pltpu.SemaphoreType.REGULAR((N_SUB,)),
            pltpu.SemaphoreType.REGULAR((N_SUB,)),
        ],
        compiler_params=pltpu.CompilerParams(
            collective_id=0, vmem_limit_bytes=62 * 1024 * 1024),
        interpret=_INTERPRET,
    )(x, w_mat, scale_x, scale_w)
